# baseline (speedup 1.0000x reference)
import functools

import jax
import jax.numpy as jnp
from jax import lax
from jax.experimental import pallas as pl
from jax.experimental.pallas import tpu as pltpu
from jax.experimental.pallas import tpu_sc as plsc

VOCAB = 1000000
MAXLEN = 200
EMBED_DIM = 64
BATCH = 4096

NUM_CORES = 2
NUM_SUBCORES = 16
NUM_WORKERS = NUM_CORES * NUM_SUBCORES
NROWS = BATCH * MAXLEN
ROWS_PER_WORKER = NROWS // NUM_WORKERS
SEQS_PER_CHUNK = 2
CHUNK = SEQS_PER_CHUNK * MAXLEN
CHUNKS_PER_WORKER = ROWS_PER_WORKER // CHUNK
SUBGATHERS = ((0, 128), (128, 128), (256, 128), (384, 16))

LANES = 16
DBLK = EMBED_DIM // LANES


@functools.partial(
    pl.kernel,
    out_type=jax.ShapeDtypeStruct((NROWS, EMBED_DIM), jnp.float32),
    mesh=plsc.VectorSubcoreMesh(core_axis_name="c", subcore_axis_name="s"),
    scratch_types=[
        pltpu.VMEM((CHUNK,), jnp.int32),
        pltpu.VMEM((CHUNK, EMBED_DIM), jnp.float32),
        pltpu.VMEM((MAXLEN, EMBED_DIM), jnp.float32),
        pltpu.SemaphoreType.DMA,
    ],
    compiler_params=pltpu.CompilerParams(use_tc_tiling_on_sc=False),
)
def _emb_kernel(x_hbm, tok_hbm, pos_hbm, out_hbm, idx_v, rows_v, pos_v, sem):
    wid = lax.axis_index("s") * NUM_CORES + lax.axis_index("c")
    base = wid * ROWS_PER_WORKER

    pltpu.sync_copy(pos_hbm, pos_v)

    def chunk_body(k, carry):
        row0 = base + k * CHUNK
        pltpu.sync_copy(x_hbm.at[pl.ds(row0, CHUNK)], idx_v)
        handles = []
        for sb, sl in SUBGATHERS:
            handles.append(
                pltpu.async_copy(
                    tok_hbm.at[idx_v.at[pl.ds(sb, sl)]],
                    rows_v.at[pl.ds(sb, sl)],
                    sem,
                )
            )
        for h in handles:
            h.wait()

        def pos_body(p, c):
            for j in range(SEQS_PER_CHUNK):
                r = j * MAXLEN + p
                for cb in range(DBLK):
                    sl16 = pl.ds(cb * LANES, LANES)
                    rows_v[r, sl16] = rows_v[r, sl16] + pos_v[p, sl16]
            return c

        lax.fori_loop(0, MAXLEN, pos_body, 0)
        pltpu.sync_copy(rows_v, out_hbm.at[pl.ds(row0, CHUNK)])
        return carry

    lax.fori_loop(0, CHUNKS_PER_WORKER, chunk_body, 0)


def kernel(x, token_emb, pos_emb):
    x_flat = x.reshape(-1).astype(jnp.int32)
    out = _emb_kernel(x_flat, token_emb, pos_emb)
    return out.reshape(BATCH, MAXLEN, EMBED_DIM)
